# baseline (speedup 1.0000x reference)
import dataclasses
import functools

import jax
import jax.numpy as jnp
from jax import lax
from jax.experimental import pallas as pl
from jax.experimental.pallas import tpu as pltpu
from jax.experimental.pallas import tpu_sc as plsc

NUM_CORES = 2
NUM_SUBCORES = 16
NUM_TILES = NUM_CORES * NUM_SUBCORES
LANES = 16
GATHER_CHUNK = 128


@functools.lru_cache(maxsize=None)
def _build(batch: int, d: int):
    assert d == 2 * LANES
    assert batch % (8 * NUM_TILES) == 0
    bpw = batch // NUM_TILES
    n_chunks = bpw // GATHER_CHUNK

    mesh = plsc.VectorSubcoreMesh(core_axis_name="c", subcore_axis_name="s")
    cp = pltpu.CompilerParams()
    if "needs_layout_passes" in pltpu.CompilerParams.__dataclass_fields__:
        cp = dataclasses.replace(cp, needs_layout_passes=False)
    if "use_tc_tiling_on_sc" in pltpu.CompilerParams.__dataclass_fields__:
        cp = dataclasses.replace(cp, use_tc_tiling_on_sc=False)

    @functools.partial(
        pl.kernel,
        out_type=jax.ShapeDtypeStruct((batch,), jnp.float32),
        mesh=mesh,
        compiler_params=cp,
        scratch_types=[
            pltpu.VMEM((bpw,), jnp.int32),
            pltpu.VMEM((bpw,), jnp.int32),
            pltpu.VMEM((bpw, d), jnp.float32),
            pltpu.VMEM((bpw, d), jnp.float32),
            pltpu.VMEM((bpw,), jnp.float32),
            pltpu.VMEM((4 * LANES,), jnp.float32),
            pltpu.VMEM((LANES,), jnp.float32),
            pltpu.SemaphoreType.DMA,
            pltpu.SemaphoreType.DMA,
        ],
    )
    def dlcrs(users_h, movies_h, ut_h, mt_h, w_h, bv_h, out_h,
              uidx, midx, urows, mrows, outv, wv, bv, sem_u, sem_m):
        wid = lax.axis_index("s") * NUM_CORES + lax.axis_index("c")
        base = wid * bpw

        pltpu.sync_copy(users_h.at[pl.ds(base, bpw)], uidx)
        pltpu.sync_copy(movies_h.at[pl.ds(base, bpw)], midx)
        pltpu.sync_copy(w_h, wv)
        pltpu.sync_copy(bv_h, bv)

        copies = []
        for c in range(n_chunks):
            sl = pl.ds(c * GATHER_CHUNK, GATHER_CHUNK)
            copies.append(
                pltpu.async_copy(ut_h.at[uidx.at[sl]], urows.at[sl], sem_u))
            copies.append(
                pltpu.async_copy(mt_h.at[midx.at[sl]], mrows.at[sl], sem_m))
        for cp in copies:
            cp.wait()

        wu0 = wv[pl.ds(0, LANES)]
        wu1 = wv[pl.ds(LANES, LANES)]
        wm0 = wv[pl.ds(2 * LANES, LANES)]
        wm1 = wv[pl.ds(3 * LANES, LANES)]
        bvv = bv[...]
        lane = lax.iota(jnp.int32, LANES)
        last = lane == (LANES - 1)

        @pl.loop(0, bpw)
        def _(r):
            u0 = urows[r, pl.ds(0, LANES)]
            u1 = urows[r, pl.ds(LANES, LANES)]
            m0 = mrows[r, pl.ds(0, LANES)]
            m1 = mrows[r, pl.ds(LANES, LANES)]
            p = u0 * wu0 + u1 * wu1 + m0 * wm0 + m1 * wm1 + bvv
            s = jnp.cumsum(p)
            idx = jnp.full((LANES,), r, jnp.int32)
            plsc.store_scatter(outv, [idx], s, mask=last)

        pltpu.sync_copy(outv, out_h.at[pl.ds(base, bpw)])

    return dlcrs


def kernel(users, movies, user_table, movie_table, W, b):
    batch = users.shape[0]
    d = user_table.shape[1]
    wflat = W.reshape(2 * d).astype(jnp.float32)
    bv = jnp.broadcast_to(b / LANES, (LANES,)).astype(jnp.float32)
    fn = _build(batch, d)
    out = fn(users.astype(jnp.int32), movies.astype(jnp.int32),
             user_table, movie_table, wflat, bv)
    return out.reshape(batch, 1)
